# baseline (speedup 1.0000x reference)
import functools

import numpy as np
import jax
import jax.numpy as jnp
from jax import lax
from jax.experimental import pallas as pl
from jax.experimental.pallas import tpu as pltpu
from jax.experimental.pallas import tpu_sc as plsc

N_NODES = 10000
N_EDGES = 320000
D = 128
D_EDGE = 16
NB = 4
S = 2500

NC = 2
NS = 16
NW = NC * NS
CH = 80
S_PAD = 2560
N_PAD = 10240
N_HALF = N_PAD // NC
TILE_ROWS = N_HALF // NS
ACC_ROWS = N_HALF + 16
DUMMY_GATH = N_HALF
DUMMY_SCAT = N_HALF + 8
DUMMY_DST = N_PAD - 8


def _sc_mesh():
    return plsc.VectorSubcoreMesh(
        core_axis_name="c", subcore_axis_name="s", num_cores=NC, num_subcores=NS
    )


def _index_prep(ei3):
    def body(a_ref, src_out, dst_out):
        a = a_ref[...].astype(jnp.float32).reshape(2 * S_PAD, 8)
        rows_i = lax.broadcasted_iota(jnp.int32, (8, NB), 0)
        cols_i = lax.broadcasted_iota(jnp.int32, (8, NB), 1)
        target = lax.rem(2 * cols_i + 2, jnp.int32(8))
        sel = (rows_i == target).astype(jnp.float32)
        both = lax.dot_general(
            sel, a, (((0,), (1,)), ((), ())),
            preferred_element_type=jnp.float32,
            precision=lax.Precision.HIGHEST,
        )
        both = both + 0.5
        srci = both[:, :S_PAD].astype(jnp.int32)
        d = both[:, S_PAD:].astype(jnp.int32)
        ci = lax.broadcasted_iota(jnp.int32, (NB, S_PAD), 1)
        d = jnp.where(ci < S, d, DUMMY_DST)
        for kk in range(NB):
            src_out[kk][...] = srci[kk]
            dst_out[kk][...] = d[kk]

    def body_wrap(a_ref, *outs):
        body(a_ref, outs[:NB], outs[NB:])

    return pl.pallas_call(
        body_wrap,
        out_shape=[jax.ShapeDtypeStruct((S_PAD,), jnp.int32)] * (2 * NB),
    )(ei3)


def _clamp_to_half(idx_v, row, c0, dummy):
    for t in range(CH // 16):
        v = idx_v[row, pl.ds(t * 16, 16)]
        lv = v - c0
        ok = (lv >= 0) & (lv < N_HALF)
        idx_v[row, pl.ds(t * 16, 16)] = jnp.where(ok, lv, dummy)


def _gather_rows(table, src1d):
    @functools.partial(
        pl.kernel,
        out_type=jax.ShapeDtypeStruct((S_PAD, D), jnp.float32),
        mesh=_sc_mesh(),
        scratch_types=[
            pltpu.VMEM((1, CH), jnp.int32),
            pltpu.VMEM((CH, D), jnp.float32),
            pltpu.SemaphoreType.DMA,
        ],
    )
    def gk(table_hbm, src_hbm, out_hbm, idx_v, rows_v, sem):
        wid = lax.axis_index("s") * NC + lax.axis_index("c")
        pltpu.sync_copy(src_hbm.at[pl.ds(wid * CH, CH)], idx_v.at[0])
        pltpu.async_copy(table_hbm.at[idx_v.at[0]], rows_v, sem).wait()
        pltpu.sync_copy(rows_v, out_hbm.at[pl.ds(wid * CH, CH)])

    return gk(table, src1d)


def _scatter_block(u, msg, dst1d, nsrc1d):
    out_type = [jax.ShapeDtypeStruct((N_PAD, D), jnp.float32)]
    scratch = [
        pltpu.VMEM_SHARED((ACC_ROWS, D), jnp.float32),
        pltpu.VMEM((2, CH), jnp.int32),
        pltpu.VMEM((2, CH, D), jnp.float32),
        pltpu.SemaphoreType.DMA,
    ]
    with_gather = nsrc1d is not None
    if with_gather:
        out_type += [
            jax.ShapeDtypeStruct((S_PAD, D), jnp.float32),
            jax.ShapeDtypeStruct((S_PAD, D), jnp.float32),
        ]
        scratch += [
            pltpu.VMEM((2, CH), jnp.int32),
            pltpu.VMEM((2 * CH, D), jnp.float32),
            pltpu.VMEM((8, D), jnp.float32),
            pltpu.SemaphoreType.DMA,
        ]

    @functools.partial(
        pl.kernel, out_type=out_type, mesh=_sc_mesh(), scratch_types=scratch
    )
    def sk(*refs):
        if with_gather:
            (u_hbm, msg_hbm, dst_hbm, nsrc_hbm, p_hbm, g0_hbm, g1_hbm,
             acc_sh, idx_v, msg_v, sem, nidx_v, grows_v, zbuf,
             gsem) = refs
        else:
            (u_hbm, msg_hbm, dst_hbm, p_hbm,
             acc_sh, idx_v, msg_v, sem) = refs
        cid = lax.axis_index("c")
        sid = lax.axis_index("s")
        c0 = cid * N_HALF
        r0 = sid * TILE_ROWS

        pltpu.sync_copy(
            u_hbm.at[pl.ds(c0 + r0, TILE_ROWS)], acc_sh.at[pl.ds(r0, TILE_ROWS)]
        )
        for j in range(2):
            j0 = sid * 2 * CH + j * CH
            pltpu.sync_copy(dst_hbm.at[pl.ds(j0, CH)], idx_v.at[j])
            _clamp_to_half(idx_v, j, c0, DUMMY_SCAT)
            pltpu.sync_copy(msg_hbm.at[pl.ds(j0, CH)], msg_v.at[j])
        if with_gather:
            for j in range(2):
                j0 = sid * 2 * CH + j * CH
                pltpu.sync_copy(nsrc_hbm.at[pl.ds(j0, CH)], nidx_v.at[j])
                _clamp_to_half(nidx_v, j, c0, DUMMY_GATH)

            @pl.when(sid == 0)
            def _zero_miss_rows():
                for r in range(8):
                    for t in range(D // 16):
                        zbuf[r, pl.ds(t * 16, 16)] = jnp.zeros((16,), jnp.float32)
                pltpu.sync_copy(zbuf, acc_sh.at[pl.ds(DUMMY_GATH, 8)])

        plsc.subcore_barrier()

        for j in range(2):
            pltpu.sync_copy(msg_v.at[j], acc_sh.at[idx_v.at[j]], add=True)

        plsc.subcore_barrier()

        wb = pltpu.async_copy(
            acc_sh.at[pl.ds(r0, TILE_ROWS)],
            p_hbm.at[pl.ds(c0 + r0, TILE_ROWS)],
            sem,
        )
        if with_gather:
            for j in range(2):
                pltpu.async_copy(
                    acc_sh.at[nidx_v.at[j]],
                    grows_v.at[pl.ds(j * CH, CH)],
                    gsem,
                ).wait()

            @pl.when(cid == 0)
            def _out0():
                pltpu.sync_copy(grows_v, g0_hbm.at[pl.ds(sid * 2 * CH, 2 * CH)])

            @pl.when(cid == 1)
            def _out1():
                pltpu.sync_copy(grows_v, g1_hbm.at[pl.ds(sid * 2 * CH, 2 * CH)])

        wb.wait()

    return (sk(u, msg, dst1d, nsrc1d) if with_gather
            else sk(u, msg, dst1d))


M_PLANES = [2, 4, 6, 0]


def _mm_block(g_parts, subea_t, p, wmsg, wedge, wself, b2, use_relu):
    n_rows = p.shape[0]
    two_g = len(g_parts) == 2

    def body(*refs):
        if two_g:
            (g0_ref, g1_ref, se_ref, p_ref, wm_ref, we_ref, ws_ref, b_ref,
             msg_out, u_out) = refs
            gg = g0_ref[...] + g1_ref[...]
        else:
            (g_ref, se_ref, p_ref, wm_ref, we_ref, ws_ref, b_ref,
             msg_out, u_out) = refs
            gg = g_ref[...]
        pp = p_ref[...]
        if use_relu:
            gg = jnp.maximum(gg, 0.0)
            pp = jnp.maximum(pp, 0.0)
        e_k = lax.dot_general(
            se_ref[...], we_ref[...], (((0,), (0,)), ((), ())),
            preferred_element_type=jnp.float32,
        )
        msg_out[...] = (
            jnp.dot(gg, wm_ref[...], preferred_element_type=jnp.float32)
            + e_k
        )
        u_out[pl.ds(0, n_rows), :] = (
            jnp.dot(pp, ws_ref[...], preferred_element_type=jnp.float32)
            + b_ref[...]
        )
        if n_rows < N_PAD:
            u_out[pl.ds(n_rows, N_PAD - n_rows), :] = jnp.zeros(
                (N_PAD - n_rows, D), jnp.float32
            )

    return pl.pallas_call(
        body,
        out_shape=[
            jax.ShapeDtypeStruct((S_PAD, D), jnp.float32),
            jax.ShapeDtypeStruct((N_PAD, D), jnp.float32),
        ],
    )(*g_parts, subea_t, p, wmsg, wedge, wself, b2)


def _relu_kernel(p):
    def body(p_ref, o_ref):
        o_ref[...] = jnp.maximum(p_ref[pl.ds(0, N_NODES), :], 0.0)

    return pl.pallas_call(
        body, out_shape=jax.ShapeDtypeStruct((N_NODES, D), jnp.float32)
    )(p)


def kernel(x, ei, ea, batch, y, W_msg, W_edge, W_self, b):
    base = np.arange(0, N_NODES, NB)
    k2 = np.stack(
        [(2 * ((base + k) % N_NODES)) % N_EDGES for k in range(1, NB + 1)]
    ).astype(np.int32)
    for ki, m in enumerate(M_PLANES):
        if not np.array_equal(np.sort(k2[ki]), np.arange(S) * 8 + m):
            raise AssertionError("static edge-subset pattern mismatch")

    ei3 = lax.slice(ei, (0, 0), (2, 8 * S_PAD)).reshape(2, S_PAD, 8)
    outs = _index_prep(ei3)
    srcs, dsts = outs[:NB], outs[NB:]
    b2 = b.reshape(1, D)

    ea_t = ea.T
    subea_t = [
        lax.slice(ea_t, (0, m), (D_EDGE, 8 * S_PAD), (1, 8))
        for m in M_PLANES
    ]

    p_cur = x
    g_parts = (_gather_rows(x, srcs[0]),)
    for k in range(NB):
        msg, u = _mm_block(g_parts, subea_t[k], p_cur, W_msg, W_edge, W_self,
                           b2, use_relu=(k > 0))
        nsrc = srcs[k + 1] if k + 1 < NB else None
        res = _scatter_block(u, msg, dsts[k], nsrc)
        if k + 1 < NB:
            p_cur, g0, g1 = res
            g_parts = (g0, g1)
        else:
            (p_cur,) = res

    return _relu_kernel(p_cur)


# speedup vs baseline: 1.0326x; 1.0326x over previous
import functools

import numpy as np
import jax
import jax.numpy as jnp
from jax import lax
from jax.experimental import pallas as pl
from jax.experimental.pallas import tpu as pltpu
from jax.experimental.pallas import tpu_sc as plsc

N_NODES = 10000
N_EDGES = 320000
D = 128
D_EDGE = 16
NB = 4
S = 2500

NC = 2
NS = 16
NW = NC * NS
CH = 80
S_PAD = 2560
N_PAD = 10240
N_HALF = N_PAD // NC
TILE_ROWS = N_HALF // NS
ACC_ROWS = N_HALF + 16
DUMMY_GATH = N_HALF
DUMMY_SCAT = N_HALF + 8
DUMMY_DST = N_PAD - 8


def _sc_mesh():
    return plsc.VectorSubcoreMesh(
        core_axis_name="c", subcore_axis_name="s", num_cores=NC, num_subcores=NS
    )


def _clamp_to_half(idx_v, row, c0, dummy):
    for t in range(CH // 16):
        v = idx_v[row, pl.ds(t * 16, 16)]
        lv = v - c0
        ok = (lv >= 0) & (lv < N_HALF)
        idx_v[row, pl.ds(t * 16, 16)] = jnp.where(ok, lv, dummy)


def _gather_rows(table, idx3):
    @functools.partial(
        pl.kernel,
        out_type=jax.ShapeDtypeStruct((S_PAD, D), jnp.float32),
        mesh=_sc_mesh(),
        scratch_types=[
            pltpu.VMEM((1, CH), jnp.int32),
            pltpu.VMEM((CH, D), jnp.float32),
            pltpu.SemaphoreType.DMA,
        ],
    )
    def gk(table_hbm, idx_hbm, out_hbm, idx_v, rows_v, sem):
        wid = lax.axis_index("s") * NC + lax.axis_index("c")
        pltpu.sync_copy(idx_hbm.at[wid], idx_v)
        pltpu.async_copy(table_hbm.at[idx_v.at[0]], rows_v, sem).wait()
        pltpu.sync_copy(rows_v, out_hbm.at[pl.ds(wid * CH, CH)])

    return gk(table, idx3)


def _scatter_block(u, msg, dst3, nsrc3):
    out_type = [jax.ShapeDtypeStruct((N_PAD, D), jnp.float32)]
    scratch = [
        pltpu.VMEM_SHARED((ACC_ROWS, D), jnp.float32),
        pltpu.VMEM((2, CH), jnp.int32),
        pltpu.VMEM((2, CH, D), jnp.float32),
        pltpu.SemaphoreType.DMA,
    ]
    with_gather = nsrc3 is not None
    if with_gather:
        out_type += [
            jax.ShapeDtypeStruct((S_PAD, D), jnp.float32),
            jax.ShapeDtypeStruct((S_PAD, D), jnp.float32),
        ]
        scratch += [
            pltpu.VMEM((2, CH), jnp.int32),
            pltpu.VMEM((2 * CH, D), jnp.float32),
            pltpu.VMEM((8, D), jnp.float32),
            pltpu.SemaphoreType.DMA,
        ]

    @functools.partial(
        pl.kernel, out_type=out_type, mesh=_sc_mesh(), scratch_types=scratch
    )
    def sk(*refs):
        if with_gather:
            (u_hbm, msg_hbm, dst_hbm, nsrc_hbm, p_hbm, g0_hbm, g1_hbm,
             acc_sh, idx_v, msg_v, sem, nidx_v, grows_v, zbuf, gsem) = refs
        else:
            (u_hbm, msg_hbm, dst_hbm, p_hbm,
             acc_sh, idx_v, msg_v, sem) = refs
        cid = lax.axis_index("c")
        sid = lax.axis_index("s")
        c0 = cid * N_HALF
        r0 = sid * TILE_ROWS

        pltpu.sync_copy(
            u_hbm.at[pl.ds(c0 + r0, TILE_ROWS)], acc_sh.at[pl.ds(r0, TILE_ROWS)]
        )
        pltpu.sync_copy(dst_hbm.at[sid], idx_v)
        for j in range(2):
            _clamp_to_half(idx_v, j, c0, DUMMY_SCAT)
            pltpu.sync_copy(
                msg_hbm.at[pl.ds(sid * 2 * CH + j * CH, CH)], msg_v.at[j]
            )
        if with_gather:
            @pl.when(sid == 0)
            def _zero_miss_rows():
                for r in range(8):
                    for t in range(D // 16):
                        zbuf[r, pl.ds(t * 16, 16)] = jnp.zeros((16,), jnp.float32)
                pltpu.sync_copy(zbuf, acc_sh.at[pl.ds(DUMMY_GATH, 8)])

        plsc.subcore_barrier()

        for j in range(2):
            pltpu.sync_copy(msg_v.at[j], acc_sh.at[idx_v.at[j]], add=True)

        plsc.subcore_barrier()

        wb = pltpu.async_copy(
            acc_sh.at[pl.ds(r0, TILE_ROWS)],
            p_hbm.at[pl.ds(c0 + r0, TILE_ROWS)],
            sem,
        )
        if with_gather:
            pltpu.sync_copy(nsrc_hbm.at[sid], nidx_v)
            for j in range(2):
                _clamp_to_half(nidx_v, j, c0, DUMMY_GATH)
                pltpu.async_copy(
                    acc_sh.at[nidx_v.at[j]],
                    grows_v.at[pl.ds(j * CH, CH)],
                    gsem,
                ).wait()

            @pl.when(cid == 0)
            def _out0():
                pltpu.sync_copy(grows_v, g0_hbm.at[pl.ds(sid * 2 * CH, 2 * CH)])

            @pl.when(cid == 1)
            def _out1():
                pltpu.sync_copy(grows_v, g1_hbm.at[pl.ds(sid * 2 * CH, 2 * CH)])

        wb.wait()

    return sk(u, msg, dst3, nsrc3) if with_gather else sk(u, msg, dst3)


M_PLANES = [2, 4, 6, 0]


def _mm_block(g_parts, subea_t, p, wmsg, wedge, wself, b2, use_relu):
    n_rows = p.shape[0]
    two_g = len(g_parts) == 2

    def body(*refs):
        if two_g:
            (g0_ref, g1_ref, se_ref, p_ref, wm_ref, we_ref, ws_ref, b_ref,
             msg_out, u_out) = refs
            gg = g0_ref[...] + g1_ref[...]
        else:
            (g_ref, se_ref, p_ref, wm_ref, we_ref, ws_ref, b_ref,
             msg_out, u_out) = refs
            gg = g_ref[...]
        pp = p_ref[...]
        if use_relu:
            gg = jnp.maximum(gg, 0.0)
            pp = jnp.maximum(pp, 0.0)
        e_k = lax.dot_general(
            se_ref[...], we_ref[...], (((0,), (0,)), ((), ())),
            preferred_element_type=jnp.float32,
        )
        msg_out[...] = (
            jnp.dot(gg, wm_ref[...], preferred_element_type=jnp.float32)
            + e_k
        )
        u_out[pl.ds(0, n_rows), :] = (
            jnp.dot(pp, ws_ref[...], preferred_element_type=jnp.float32)
            + b_ref[...]
        )
        if n_rows < N_PAD:
            u_out[pl.ds(n_rows, N_PAD - n_rows), :] = jnp.zeros(
                (N_PAD - n_rows, D), jnp.float32
            )

    return pl.pallas_call(
        body,
        out_shape=[
            jax.ShapeDtypeStruct((S_PAD, D), jnp.float32),
            jax.ShapeDtypeStruct((N_PAD, D), jnp.float32),
        ],
    )(*g_parts, subea_t, p, wmsg, wedge, wself, b2)


def _relu_kernel(p):
    def body(p_ref, o_ref):
        o_ref[...] = jnp.maximum(p_ref[pl.ds(0, N_NODES), :], 0.0)

    return pl.pallas_call(
        body, out_shape=jax.ShapeDtypeStruct((N_NODES, D), jnp.float32)
    )(p)


def kernel(x, ei, ea, batch, y, W_msg, W_edge, W_self, b):
    base = np.arange(0, N_NODES, NB)
    k2 = np.stack(
        [(2 * ((base + k) % N_NODES)) % N_EDGES for k in range(1, NB + 1)]
    ).astype(np.int32)
    for ki, m in enumerate(M_PLANES):
        if not np.array_equal(np.sort(k2[ki]), np.arange(S) * 8 + m):
            raise AssertionError("static edge-subset pattern mismatch")

    cols = lax.slice(ei, (0, 0), (2, 8 * S)).reshape(2, S, 8)
    srcs, dsts = [], []
    pad_i = jnp.zeros((S_PAD - S,), jnp.int32)
    pad_d = jnp.full((S_PAD - S,), DUMMY_DST, jnp.int32)
    for m in M_PLANES:
        srcs.append(jnp.concatenate([cols[0, :, m], pad_i]))
        dsts.append(jnp.concatenate([cols[1, :, m], pad_d]))
    src = jnp.stack(srcs)
    dst = jnp.stack(dsts)

    src_w = src.reshape(NB, NW, 1, CH)
    src_t = src.reshape(NB, NS, 2, CH)
    dst3 = dst.reshape(NB, NS, 2, CH)
    b2 = b.reshape(1, D)

    ea_t = ea.T
    subea_t = [
        lax.slice(ea_t, (0, m), (D_EDGE, 8 * S_PAD), (1, 8))
        for m in M_PLANES
    ]

    p_cur = x
    g_parts = (_gather_rows(x, src_w[0]),)
    for k in range(NB):
        msg, u = _mm_block(g_parts, subea_t[k], p_cur, W_msg, W_edge, W_self,
                           b2, use_relu=(k > 0))
        nsrc = src_t[k + 1] if k + 1 < NB else None
        res = _scatter_block(u, msg, dst3[k], nsrc)
        if k + 1 < NB:
            p_cur, g0, g1 = res
            g_parts = (g0, g1)
        else:
            (p_cur,) = res

    return _relu_kernel(p_cur)
